# baseline (speedup 1.0000x reference)
import functools

import jax
import jax.numpy as jnp
from jax import lax
from jax.experimental import pallas as pl
from jax.experimental.pallas import tpu as pltpu
from jax.experimental.pallas import tpu_sc as plsc

N, D, K = 4096, 64, 512
BLK = 512
NH = N // 2
NW = 32
RPW = NH // NW
CH = 32
NCH = RPW // CH
BIG = 1 << 30


def _tc_dist_body(x_ref, c_ref, d2_ref, out_min_ref):
    xb = x_ref[...]
    c = c_ref[...]
    g = lax.dot_general(
        xb, c, (((1,), (1,)), ((), ())),
        preferred_element_type=jnp.float32,
        precision=lax.Precision.HIGHEST,
    )
    xn = jnp.sum(xb * xb, axis=1)
    cn = jnp.sum(c * c, axis=1)
    d2 = jnp.maximum(xn[:, None] + cn[None, :] - 2.0 * g, 0.0)
    d2_ref[...] = d2
    out_min_ref[...] = jnp.sqrt(jnp.min(d2, axis=1))


def _tc_dist(x_half, centers):
    return pl.pallas_call(
        _tc_dist_body,
        grid=(NH // BLK,),
        in_specs=[
            pl.BlockSpec((BLK, D), lambda i: (i, 0)),
            pl.BlockSpec((K, D), lambda i: (0, 0)),
        ],
        out_specs=[
            pl.BlockSpec((BLK, K), lambda i: (i, 0)),
            pl.BlockSpec((BLK,), lambda i: (i,)),
        ],
        out_shape=[
            jax.ShapeDtypeStruct((NH, K), jnp.float32),
            jax.ShapeDtypeStruct((NH,), jnp.float32),
        ],
        compiler_params=pltpu.CompilerParams(
            dimension_semantics=("arbitrary",),
        ),
    )(x_half, centers)


def _sc_colmin_body(d2_hbm, bval_hbm, bidx_hbm, buf_v, bv_v, bi_v, sem_a, sem_b):
    cid = lax.axis_index("c")
    sid = lax.axis_index("s")
    wid = sid * 2 + cid
    base = wid * RPW

    inf16 = jnp.full((16,), jnp.inf, dtype=jnp.float32)
    n16 = jnp.full((16,), BIG, dtype=jnp.int32)

    def init_loop(j, _):
        bv_v[pl.ds(j * 16, 16)] = inf16
        bi_v[pl.ds(j * 16, 16)] = n16
        return 0

    lax.fori_loop(0, K // 16, init_loop, 0)

    sems = (sem_a, sem_b)
    copies = [None, None]
    copies[0] = pltpu.async_copy(
        d2_hbm.at[pl.ds(base, CH), :], buf_v.at[0], sems[0]
    )
    for ch in range(NCH):
        cur = ch % 2
        nxt = (ch + 1) % 2
        if ch + 1 < NCH:
            copies[nxt] = pltpu.async_copy(
                d2_hbm.at[pl.ds(base + (ch + 1) * CH, CH), :],
                buf_v.at[nxt],
                sems[nxt],
            )
        copies[cur].wait()
        p0 = base + ch * CH

        def jg_loop(jg, _, cur=cur, p0=p0):
            col = jg * 64
            bvs = tuple(bv_v[pl.ds(col + c * 16, 16)] for c in range(4))
            bis = tuple(bi_v[pl.ds(col + c * 16, 16)] for c in range(4))

            def p_loop(p, carry):
                cbv, cbi = carry
                idx = jnp.full((16,), p0 + p, dtype=jnp.int32)
                nbv, nbi = [], []
                for c in range(4):
                    v = buf_v[cur, p, pl.ds(col + c * 16, 16)]
                    m = v < cbv[c]
                    nbv.append(jnp.where(m, v, cbv[c]))
                    nbi.append(jnp.where(m, idx, cbi[c]))
                return tuple(nbv), tuple(nbi)

            bvs, bis = lax.fori_loop(0, CH, p_loop, (bvs, bis))
            for c in range(4):
                bv_v[pl.ds(col + c * 16, 16)] = bvs[c]
                bi_v[pl.ds(col + c * 16, 16)] = bis[c]
            return 0

        lax.fori_loop(0, K // 64, jg_loop, 0)

    pltpu.sync_copy(bv_v, bval_hbm.at[wid])
    pltpu.sync_copy(bi_v, bidx_hbm.at[wid])


@functools.partial(
    pl.kernel,
    out_type=[
        jax.ShapeDtypeStruct((NW, K), jnp.float32),
        jax.ShapeDtypeStruct((NW, K), jnp.int32),
    ],
    mesh=plsc.VectorSubcoreMesh(core_axis_name="c", subcore_axis_name="s"),
    scratch_types=[
        pltpu.VMEM((2, CH, K), jnp.float32),
        pltpu.VMEM((K,), jnp.float32),
        pltpu.VMEM((K,), jnp.int32),
        pltpu.SemaphoreType.DMA,
        pltpu.SemaphoreType.DMA,
    ],
)
def _sc_colmin(d2_hbm, bval_hbm, bidx_hbm, buf_v, bv_v, bi_v, sem_a, sem_b):
    _sc_colmin_body(d2_hbm, bval_hbm, bidx_hbm, buf_v, bv_v, bi_v, sem_a, sem_b)


def _tc_merge_body(bval0_ref, bidx0_ref, bval1_ref, bidx1_ref, out_idx_ref):
    bval0 = bval0_ref[...]
    bval1 = bval1_ref[...]
    minv = jnp.minimum(jnp.min(bval0, axis=0), jnp.min(bval1, axis=0))
    idx0 = jnp.min(
        jnp.where(bval0 == minv[None, :], bidx0_ref[...], BIG), axis=0
    )
    idx1 = jnp.min(
        jnp.where(bval1 == minv[None, :], bidx1_ref[...] + NH, BIG), axis=0
    )
    out_idx_ref[...] = jnp.minimum(idx0, idx1)


def _tc_merge(bval0, bidx0, bval1, bidx1):
    return pl.pallas_call(
        _tc_merge_body,
        out_shape=jax.ShapeDtypeStruct((K,), jnp.int32),
    )(bval0, bidx0, bval1, bidx1)


@jax.jit
def kernel(x, centers):
    d2_0, min0 = _tc_dist(x[:NH], centers)
    bval0, bidx0 = _sc_colmin(d2_0)
    d2_1, min1 = _tc_dist(x[NH:], centers)
    bval1, bidx1 = _sc_colmin(d2_1)
    out_idx = _tc_merge(bval0, bidx0, bval1, bidx1)
    return out_idx, jnp.concatenate([min0, min1]), centers


# speedup vs baseline: 1.0108x; 1.0108x over previous
import functools

import jax
import jax.numpy as jnp
from jax import lax
from jax.experimental import pallas as pl
from jax.experimental.pallas import tpu as pltpu
from jax.experimental.pallas import tpu_sc as plsc

N, D, K = 4096, 64, 512
BLK = 1024
NW = 32
RPW = N // NW
CH = RPW // 2
BIG = 1 << 30


def _tc_dist_body(x_ref, c_ref, d2_ref, out_min_ref):
    xb = x_ref[...]
    c = c_ref[...]
    g = lax.dot_general(
        xb, c, (((1,), (1,)), ((), ())),
        preferred_element_type=jnp.float32,
        precision=lax.Precision.HIGHEST,
    )
    xn = jnp.sum(xb * xb, axis=1)
    cn = jnp.sum(c * c, axis=1)
    d2 = xn[:, None] + cn[None, :] - 2.0 * g
    d2_ref[...] = d2
    out_min_ref[...] = jnp.sqrt(jnp.maximum(jnp.min(d2, axis=1), 0.0))


def _tc_dist(x, centers):
    return pl.pallas_call(
        _tc_dist_body,
        grid=(N // BLK,),
        in_specs=[
            pl.BlockSpec((BLK, D), lambda i: (i, 0)),
            pl.BlockSpec((K, D), lambda i: (0, 0)),
        ],
        out_specs=[
            pl.BlockSpec((BLK, K), lambda i: (i, 0)),
            pl.BlockSpec((BLK,), lambda i: (i,)),
        ],
        out_shape=[
            jax.ShapeDtypeStruct((N, K), jnp.float32),
            jax.ShapeDtypeStruct((N,), jnp.float32),
        ],
        compiler_params=pltpu.CompilerParams(
            dimension_semantics=("arbitrary",),
        ),
    )(x, centers)


def _sc_colmin_body(d2_hbm, bval_hbm, bidx_hbm, buf_v, bv_v, bi_v, sem_a, sem_b):
    cid = lax.axis_index("c")
    sid = lax.axis_index("s")
    wid = sid * 2 + cid
    base = wid * RPW

    copy0 = pltpu.async_copy(
        d2_hbm.at[pl.ds(base, CH), :], buf_v.at[0], sem_a
    )
    copy1 = pltpu.async_copy(
        d2_hbm.at[pl.ds(base + CH, CH), :], buf_v.at[1], sem_b
    )

    def scan_chunk(cur, p0, init_from_vmem):
        def jg_loop(jg, _):
            col = jg * 64
            if init_from_vmem:
                bvs = tuple(bv_v[pl.ds(col + c * 16, 16)] for c in range(4))
                bis = tuple(bi_v[pl.ds(col + c * 16, 16)] for c in range(4))
            else:
                bvs = (jnp.full((16,), jnp.inf, dtype=jnp.float32),) * 4
                bis = (jnp.full((16,), BIG, dtype=jnp.int32),) * 4

            def p_loop(pp, carry):
                cbv, cbi = carry
                cbv, cbi = list(cbv), list(cbi)
                for u in range(2):
                    p = pp * 2 + u
                    idx = jnp.full((16,), p0 + p, dtype=jnp.int32)
                    for c in range(4):
                        v = buf_v[cur, p, pl.ds(col + c * 16, 16)]
                        m = v < cbv[c]
                        cbv[c] = jnp.minimum(v, cbv[c])
                        cbi[c] = jnp.where(m, idx, cbi[c])
                return tuple(cbv), tuple(cbi)

            bvs, bis = lax.fori_loop(0, CH // 2, p_loop, (bvs, bis))
            for c in range(4):
                bv_v[pl.ds(col + c * 16, 16)] = bvs[c]
                bi_v[pl.ds(col + c * 16, 16)] = bis[c]
            return 0

        lax.fori_loop(0, K // 64, jg_loop, 0)

    copy0.wait()
    scan_chunk(0, base, False)
    copy1.wait()
    scan_chunk(1, base + CH, True)

    pltpu.sync_copy(bv_v, bval_hbm.at[wid])
    pltpu.sync_copy(bi_v, bidx_hbm.at[wid])


@functools.partial(
    pl.kernel,
    out_type=[
        jax.ShapeDtypeStruct((NW, K), jnp.float32),
        jax.ShapeDtypeStruct((NW, K), jnp.int32),
    ],
    mesh=plsc.VectorSubcoreMesh(core_axis_name="c", subcore_axis_name="s"),
    scratch_types=[
        pltpu.VMEM((2, CH, K), jnp.float32),
        pltpu.VMEM((K,), jnp.float32),
        pltpu.VMEM((K,), jnp.int32),
        pltpu.SemaphoreType.DMA,
        pltpu.SemaphoreType.DMA,
    ],
)
def _sc_colmin(d2_hbm, bval_hbm, bidx_hbm, buf_v, bv_v, bi_v, sem_a, sem_b):
    _sc_colmin_body(d2_hbm, bval_hbm, bidx_hbm, buf_v, bv_v, bi_v, sem_a, sem_b)


def _tc_merge_body(bval_ref, bidx_ref, out_idx_ref):
    bval = bval_ref[...]
    bidx = bidx_ref[...]
    minv = jnp.min(bval, axis=0)
    out_idx_ref[...] = jnp.min(
        jnp.where(bval == minv[None, :], bidx, BIG), axis=0
    )


def _tc_merge(bval, bidx):
    return pl.pallas_call(
        _tc_merge_body,
        out_shape=jax.ShapeDtypeStruct((K,), jnp.int32),
    )(bval, bidx)


@jax.jit
def kernel(x, centers):
    d2, out_min = _tc_dist(x, centers)
    bval, bidx = _sc_colmin(d2)
    out_idx = _tc_merge(bval, bidx)
    return out_idx, out_min, centers


# speedup vs baseline: 1.2567x; 1.2433x over previous
import functools

import jax
import jax.numpy as jnp
from jax import lax
from jax.experimental import pallas as pl
from jax.experimental.pallas import tpu as pltpu
from jax.experimental.pallas import tpu_sc as plsc

N, D, K = 4096, 64, 512
BLKN = 1024
NW = 32
RPS = K // NW
HC = N // 2
BIG = 1 << 30


def _tc_dist_body(xt_ref, ct_ref, d2t_ref, out_min_ref):
    xb = xt_ref[...]
    ct = ct_ref[...]
    g = lax.dot_general(
        ct, xb, (((0,), (0,)), ((), ())),
        preferred_element_type=jnp.float32,
        precision=lax.Precision.HIGHEST,
    )
    cn = jnp.sum(ct * ct, axis=0)
    xn = jnp.sum(xb * xb, axis=0)
    d2t = cn[:, None] + xn[None, :] - 2.0 * g
    d2t_ref[...] = d2t
    out_min_ref[...] = jnp.sqrt(jnp.maximum(jnp.min(d2t, axis=0), 0.0))


def _tc_dist(xt, ct):
    return pl.pallas_call(
        _tc_dist_body,
        grid=(N // BLKN,),
        in_specs=[
            pl.BlockSpec((D, BLKN), lambda i: (0, i)),
            pl.BlockSpec((D, K), lambda i: (0, 0)),
        ],
        out_specs=[
            pl.BlockSpec((K, BLKN), lambda i: (0, i)),
            pl.BlockSpec((BLKN,), lambda i: (i,)),
        ],
        out_shape=[
            jax.ShapeDtypeStruct((K, N), jnp.float32),
            jax.ShapeDtypeStruct((N,), jnp.float32),
        ],
        compiler_params=pltpu.CompilerParams(
            dimension_semantics=("arbitrary",),
        ),
    )(xt, ct)


def _sc_argmin_body(d2t_hbm, oidx_hbm, buf_v, sbv_v, sbi_v, out_v, sem_a, sem_b):
    cid = lax.axis_index("c")
    sid = lax.axis_index("s")
    wid = sid * 2 + cid
    r0 = wid * RPS

    copy0 = pltpu.async_copy(
        d2t_hbm.at[pl.ds(r0, RPS), pl.ds(0, HC)], buf_v.at[0], sem_a
    )
    copy1 = pltpu.async_copy(
        d2t_hbm.at[pl.ds(r0, RPS), pl.ds(HC, HC)], buf_v.at[1], sem_b
    )
    iota16 = lax.iota(jnp.int32, 16)

    def lane_gather(a, perm):
        dn = lax.GatherDimensionNumbers(
            offset_dims=(), collapsed_slice_dims=(0,), start_index_map=(0,))
        return lax.gather(a, perm[:, None], dn, (1,),
                          mode=lax.GatherScatterMode.PROMISE_IN_BOUNDS)

    def scan_row(cur, r, bv, bi, tbase):
        def t_loop(t, carry):
            cbv, cbi = carry
            for u in range(4):
                tt = t * 4 + u
                v = buf_v[cur, r, pl.ds(tt * 16, 16)]
                tb = jnp.full((16,), tbase + tt, dtype=jnp.int32)
                m = v < cbv
                cbv = jnp.minimum(v, cbv)
                cbi = jnp.where(m, tb, cbi)
            return cbv, cbi

        return lax.fori_loop(0, HC // 64, t_loop, (bv, bi))

    copy0.wait()

    def row_a(r, _):
        bv = jnp.full((16,), jnp.inf, dtype=jnp.float32)
        bi = jnp.full((16,), BIG, dtype=jnp.int32)
        bv, bi = scan_row(0, r, bv, bi, 0)
        sbv_v[r, :] = bv
        sbi_v[r, :] = bi
        return 0

    lax.fori_loop(0, RPS, row_a, 0)
    copy1.wait()

    def row_b(r, outv):
        bv, bi = scan_row(1, r, sbv_v[r, :], sbi_v[r, :], HC // 16)
        bi = bi * 16 + iota16
        for s in (8, 4, 2, 1):
            perm = jnp.bitwise_xor(iota16, s)
            ov = lane_gather(bv, perm)
            oi = lane_gather(bi, perm)
            take = (ov < bv) | ((ov == bv) & (oi < bi))
            bv = jnp.where(take, ov, bv)
            bi = jnp.where(take, oi, bi)
        return jnp.where(iota16 == r, bi, outv)

    outv = lax.fori_loop(0, RPS, row_b, jnp.full((16,), BIG, dtype=jnp.int32))
    out_v[...] = outv
    pltpu.sync_copy(out_v, oidx_hbm.at[pl.ds(r0, RPS)])


@functools.partial(
    pl.kernel,
    out_type=jax.ShapeDtypeStruct((K,), jnp.int32),
    mesh=plsc.VectorSubcoreMesh(core_axis_name="c", subcore_axis_name="s"),
    scratch_types=[
        pltpu.VMEM((2, RPS, HC), jnp.float32),
        pltpu.VMEM((RPS, 16), jnp.float32),
        pltpu.VMEM((RPS, 16), jnp.int32),
        pltpu.VMEM((16,), jnp.int32),
        pltpu.SemaphoreType.DMA,
        pltpu.SemaphoreType.DMA,
    ],
)
def _sc_argmin(d2t_hbm, oidx_hbm, buf_v, sbv_v, sbi_v, out_v, sem_a, sem_b):
    _sc_argmin_body(d2t_hbm, oidx_hbm, buf_v, sbv_v, sbi_v, out_v, sem_a, sem_b)


@jax.jit
def kernel(x, centers):
    d2t, out_min = _tc_dist(x.T, centers.T)
    out_idx = _sc_argmin(d2t)
    return out_idx, out_min, centers
